# baseline (speedup 1.0000x reference)
import functools

import jax
import jax.numpy as jnp
from jax import lax
from jax.experimental import pallas as pl
from jax.experimental.pallas import tpu as pltpu
from jax.experimental.pallas import tpu_sc as plsc

N, E, C, H = 10000, 320000, 128, 16

_info = plsc.get_sparse_core_info()
NC, NS = _info.num_cores, _info.num_subcores
NW = NC * NS
CH = 128
NCHUNK = E // CH
ECB = NCHUNK // NW
NEXTRA = NCHUNK - ECB * NW
RPS = N // NS
FR = N * H // 128
CB = 3
NG = ECB // CB
NBUF = 8
NGM = (NG // NBUF) * NBUF
GR = CB * CH


def _mm1_body(x_ref, w_ref, y_ref):
    y_ref[...] = jnp.dot(x_ref[...], w_ref[...],
                         preferred_element_type=jnp.float32)


def _mlp_body(y_ref, p_ref, s_ref, b1_ref, w2_ref, b2_ref, o_ref):
    h = s_ref[0, 0] * y_ref[...] + p_ref[0] + p_ref[1] + b1_ref[...]
    h = jnp.maximum(h, 0.0)
    h = jnp.dot(h, w2_ref[...], preferred_element_type=jnp.float32) + b2_ref[...]
    o_ref[...] = jnp.maximum(h, 0.0)


def _make_scatter():
    mesh = plsc.VectorSubcoreMesh(core_axis_name="c", subcore_axis_name="s")

    @functools.partial(
        pl.kernel,
        mesh=mesh,
        out_type=jax.ShapeDtypeStruct((NC, N, H), jnp.float32),
        compiler_params=pltpu.CompilerParams(use_tc_tiling_on_sc=False),
        scratch_types=[
            pltpu.VMEM((ECB * CH + CH,), jnp.int32),
            pltpu.VMEM((ECB * CH + CH,), jnp.int32),
            [pltpu.VMEM((GR, H), jnp.float32)] * NBUF,
            pltpu.VMEM_SHARED((N, H), jnp.float32),
            pltpu.VMEM_SHARED((N, H), jnp.float32),
            [pltpu.SemaphoreType.DMA] * NBUF,
            [pltpu.SemaphoreType.DMA] * NBUF,
        ],
    )
    def scatter_k(ei_hbm, y_hbm, out_hbm,
                  src_v, dst_v, rows, aggr, y_spm, gsem, ssem):
        c = lax.axis_index("c")
        s = lax.axis_index("s")
        wid = s * NC + c

        epb = ECB * CH
        base = wid * epb
        pltpu.sync_copy(ei_hbm.at[0, pl.ds(base, epb)], src_v.at[pl.ds(0, epb)])
        pltpu.sync_copy(ei_hbm.at[1, pl.ds(base, epb)], dst_v.at[pl.ds(0, epb)])

        @pl.when(wid >= NW - NEXTRA)
        def _extra_idx():
            xb = epb * NW + (wid - (NW - NEXTRA)) * CH
            pltpu.sync_copy(ei_hbm.at[0, pl.ds(xb, CH)],
                            src_v.at[pl.ds(epb, CH)])
            pltpu.sync_copy(ei_hbm.at[1, pl.ds(xb, CH)],
                            dst_v.at[pl.ds(epb, CH)])

        pltpu.sync_copy(y_hbm.at[pl.ds(s * RPS, RPS)],
                        y_spm.at[pl.ds(s * RPS, RPS)])

        z = jnp.zeros((H,), jnp.float32)

        def zb(i, carry):
            rows[0][i, :] = z
            return carry

        lax.fori_loop(0, RPS, zb, 0)
        pltpu.sync_copy(rows[0].at[pl.ds(0, RPS)],
                        aggr.at[pl.ds(s * RPS, RPS)])

        plsc.subcore_barrier()

        @pl.when(wid >= NW - NEXTRA)
        def _extra_chunk():
            pltpu.async_copy(y_spm.at[src_v.at[pl.ds(epb, CH)]],
                             rows[0].at[pl.ds(0, CH)], gsem[0]).wait()
            pltpu.sync_copy(rows[0].at[pl.ds(0, CH)],
                            aggr.at[dst_v.at[pl.ds(epb, CH)]], add=True)

        for b in range(NBUF):
            pltpu.async_copy(y_spm.at[src_v.at[pl.ds(b * GR, GR)]],
                             rows[b], gsem[b])

        def body(i, carry):
            g0 = NBUF * i
            for b in range(NBUF):
                g = g0 + b
                pltpu.make_async_copy(
                    y_spm.at[src_v.at[pl.ds(g * GR, GR)]],
                    rows[b], gsem[b]).wait()
                pltpu.async_copy(rows[b],
                                 aggr.at[dst_v.at[pl.ds(g * GR, GR)]],
                                 ssem[b], add=True)
            for b in range(NBUF):
                g = g0 + b
                pltpu.make_async_copy(
                    rows[b], aggr.at[dst_v.at[pl.ds(g * GR, GR)]],
                    ssem[b]).wait()

                @pl.when(g + NBUF < NGM)
                def _fire_next():
                    pltpu.async_copy(
                        y_spm.at[src_v.at[pl.ds((g + NBUF) * GR, GR)]],
                        rows[b], gsem[b])
            return carry

        lax.fori_loop(0, NGM // NBUF, body, 0)

        for g in range(NGM, NG):
            pltpu.async_copy(y_spm.at[src_v.at[pl.ds(g * GR, GR)]],
                             rows[0], gsem[0]).wait()
            pltpu.sync_copy(rows[0], aggr.at[dst_v.at[pl.ds(g * GR, GR)]],
                            add=True)

        plsc.subcore_barrier()

        pltpu.sync_copy(aggr.at[pl.ds(s * RPS, RPS)],
                        out_hbm.at[c, pl.ds(s * RPS, RPS)])

    return scatter_k


_scatter_k = _make_scatter()


def kernel(x, edge_index, eps, W1, b1, W2, b2):
    assert x.shape == (N, C) and edge_index.shape == (2, E)

    y = pl.pallas_call(
        _mm1_body,
        out_shape=jax.ShapeDtypeStruct((N, H), jnp.float32),
    )(x, W1)

    partials = _scatter_k(edge_index, y)

    p_flat = partials.reshape(NC, FR, 128)
    scale = (1.0 + eps).reshape(1, 1).astype(jnp.float32)
    b1t = jnp.tile(b1, 128 // H).reshape(1, 128)
    b2t = jnp.tile(b2, 128 // H).reshape(1, 128)
    w2bd = jnp.kron(jnp.eye(128 // H, dtype=jnp.float32), W2)
    o_flat = pl.pallas_call(
        _mlp_body,
        out_shape=jax.ShapeDtypeStruct((FR, 128), jnp.float32),
    )(y.reshape(FR, 128), p_flat, scale, b1t, w2bd, b2t)
    return o_flat.reshape(N, H)


# speedup vs baseline: 1.0169x; 1.0169x over previous
import functools

import jax
import jax.numpy as jnp
from jax import lax
from jax.experimental import pallas as pl
from jax.experimental.pallas import tpu as pltpu
from jax.experimental.pallas import tpu_sc as plsc

N, E, C, H = 10000, 320000, 128, 16

_info = plsc.get_sparse_core_info()
NC, NS = _info.num_cores, _info.num_subcores
NW = NC * NS
CH = 128
NCHUNK = E // CH
ECB = NCHUNK // NW
NEXTRA = NCHUNK - ECB * NW
RPS = N // NS
FR = N * H // 128
CB = 6
NG = ECB // CB
NBUF = 6
NGM = (NG // NBUF) * NBUF
GR = CB * CH


def _mm1_body(x_ref, w_ref, y_ref):
    y_ref[...] = jnp.dot(x_ref[...], w_ref[...],
                         preferred_element_type=jnp.float32)


def _mlp_body(y_ref, p_ref, s_ref, b1_ref, w2_ref, b2_ref, o_ref):
    h = s_ref[0, 0] * y_ref[...] + p_ref[0] + p_ref[1] + b1_ref[...]
    h = jnp.maximum(h, 0.0)
    h = jnp.dot(h, w2_ref[...], preferred_element_type=jnp.float32) + b2_ref[...]
    o_ref[...] = jnp.maximum(h, 0.0)


def _make_scatter():
    mesh = plsc.VectorSubcoreMesh(core_axis_name="c", subcore_axis_name="s")

    @functools.partial(
        pl.kernel,
        mesh=mesh,
        out_type=jax.ShapeDtypeStruct((NC, N, H), jnp.float32),
        compiler_params=pltpu.CompilerParams(use_tc_tiling_on_sc=False),
        scratch_types=[
            pltpu.VMEM((ECB * CH + CH,), jnp.int32),
            pltpu.VMEM((ECB * CH + CH,), jnp.int32),
            [pltpu.VMEM((GR, H), jnp.float32)] * NBUF,
            pltpu.VMEM_SHARED((N, H), jnp.float32),
            pltpu.VMEM_SHARED((N, H), jnp.float32),
            [pltpu.SemaphoreType.DMA] * NBUF,
            [pltpu.SemaphoreType.DMA] * NBUF,
        ],
    )
    def scatter_k(ei_hbm, y_hbm, out_hbm,
                  src_v, dst_v, rows, aggr, y_spm, gsem, ssem):
        c = lax.axis_index("c")
        s = lax.axis_index("s")
        wid = s * NC + c

        epb = ECB * CH
        base = wid * epb
        pltpu.sync_copy(ei_hbm.at[0, pl.ds(base, epb)], src_v.at[pl.ds(0, epb)])
        pltpu.sync_copy(ei_hbm.at[1, pl.ds(base, epb)], dst_v.at[pl.ds(0, epb)])

        @pl.when(wid >= NW - NEXTRA)
        def _extra_idx():
            xb = epb * NW + (wid - (NW - NEXTRA)) * CH
            pltpu.sync_copy(ei_hbm.at[0, pl.ds(xb, CH)],
                            src_v.at[pl.ds(epb, CH)])
            pltpu.sync_copy(ei_hbm.at[1, pl.ds(xb, CH)],
                            dst_v.at[pl.ds(epb, CH)])

        pltpu.sync_copy(y_hbm.at[pl.ds(s * RPS, RPS)],
                        y_spm.at[pl.ds(s * RPS, RPS)])

        z = jnp.zeros((H,), jnp.float32)

        def zb(i, carry):
            rows[0][i, :] = z
            return carry

        lax.fori_loop(0, RPS, zb, 0)
        pltpu.sync_copy(rows[0].at[pl.ds(0, RPS)],
                        aggr.at[pl.ds(s * RPS, RPS)])

        plsc.subcore_barrier()

        @pl.when(wid >= NW - NEXTRA)
        def _extra_chunk():
            pltpu.async_copy(y_spm.at[src_v.at[pl.ds(epb, CH)]],
                             rows[0].at[pl.ds(0, CH)], gsem[0]).wait()
            pltpu.sync_copy(rows[0].at[pl.ds(0, CH)],
                            aggr.at[dst_v.at[pl.ds(epb, CH)]], add=True)

        for b in range(NBUF):
            pltpu.async_copy(y_spm.at[src_v.at[pl.ds(b * GR, GR)]],
                             rows[b], gsem[b])

        def body(i, carry):
            g0 = NBUF * i
            for b in range(NBUF):
                g = g0 + b
                pltpu.make_async_copy(
                    y_spm.at[src_v.at[pl.ds(g * GR, GR)]],
                    rows[b], gsem[b]).wait()
                pltpu.async_copy(rows[b],
                                 aggr.at[dst_v.at[pl.ds(g * GR, GR)]],
                                 ssem[b], add=True)
            for b in range(NBUF):
                g = g0 + b
                pltpu.make_async_copy(
                    rows[b], aggr.at[dst_v.at[pl.ds(g * GR, GR)]],
                    ssem[b]).wait()

                @pl.when(g + NBUF < NGM)
                def _fire_next():
                    pltpu.async_copy(
                        y_spm.at[src_v.at[pl.ds((g + NBUF) * GR, GR)]],
                        rows[b], gsem[b])
            return carry

        lax.fori_loop(0, NGM // NBUF, body, 0)

        for g in range(NGM, NG):
            pltpu.async_copy(y_spm.at[src_v.at[pl.ds(g * GR, GR)]],
                             rows[0], gsem[0]).wait()
            pltpu.sync_copy(rows[0], aggr.at[dst_v.at[pl.ds(g * GR, GR)]],
                            add=True)

        plsc.subcore_barrier()

        pltpu.sync_copy(aggr.at[pl.ds(s * RPS, RPS)],
                        out_hbm.at[c, pl.ds(s * RPS, RPS)])

    return scatter_k


_scatter_k = _make_scatter()


def kernel(x, edge_index, eps, W1, b1, W2, b2):
    assert x.shape == (N, C) and edge_index.shape == (2, E)

    y = pl.pallas_call(
        _mm1_body,
        out_shape=jax.ShapeDtypeStruct((N, H), jnp.float32),
    )(x, W1)

    partials = _scatter_k(edge_index, y)

    p_flat = partials.reshape(NC, FR, 128)
    scale = (1.0 + eps).reshape(1, 1).astype(jnp.float32)
    b1t = jnp.tile(b1, 128 // H).reshape(1, 128)
    b2t = jnp.tile(b2, 128 // H).reshape(1, 128)
    w2bd = jnp.kron(jnp.eye(128 // H, dtype=jnp.float32), W2)
    o_flat = pl.pallas_call(
        _mlp_body,
        out_shape=jax.ShapeDtypeStruct((FR, 128), jnp.float32),
    )(y.reshape(FR, 128), p_flat, scale, b1t, w2bd, b2t)
    return o_flat.reshape(N, H)
